# baseline (speedup 1.0000x reference)
import functools

import jax
import jax.numpy as jnp
from jax import lax
from jax.experimental import pallas as pl
from jax.experimental.pallas import tpu as pltpu
from jax.experimental.pallas import tpu_sc as plsc

VOCAB = 100000
HIDDEN = 1024
MAX_POS = 4096
BATCH = 4
SEQ = 4096
EPS = 1e-12

NC = 2
NS = 16
L = 16
NW = NC * NS
P_PER_W = SEQ // NW
K = 16
N_CHUNKS = P_PER_W // K
NB = N_CHUNKS * BATCH
NV = HIDDEN // L
R = 4
NBUF = 4


def _lane_sum(v):
    idx = lax.iota(jnp.int32, L)
    dnums = lax.GatherDimensionNumbers(
        offset_dims=(), collapsed_slice_dims=(0,), start_index_map=(0,))
    for d in (1, 2, 4, 8):
        perm = lax.gather(
            v, (idx ^ d)[:, None], dnums, slice_sizes=(1,),
            mode=lax.GatherScatterMode.PROMISE_IN_BOUNDS)
        v = v + perm
    return v


def _rsqrt(x):
    yi = lax.bitcast_convert_type(x, jnp.int32)
    yi = 0x5F3759DF - lax.shift_right_logical(yi, 1)
    r = lax.bitcast_convert_type(yi, jnp.float32)
    half_x = 0.5 * x
    for _ in range(3):
        r = r * (1.5 - half_x * r * r)
    return r


def _ln_quad(rows, posb, ebuf, gbuf, bbuf, base, pbase, inv_n, q, _carry):
    r0 = base + q * R
    p0 = pbase + q * R
    e0 = q * R
    zero = jnp.zeros((L,), jnp.float32)

    @plsc.parallel_loop(0, NV, carry=(zero,) * (2 * R), unroll=4)
    def acc(j, carry):
        carry = list(carry)
        for i in range(R):
            e = rows[r0 + i, pl.ds(j * L, L)] + posb[p0 + i, pl.ds(j * L, L)]
            ebuf[e0 + i, pl.ds(j * L, L)] = e
            carry[i] = carry[i] + e
            carry[R + i] = carry[R + i] + e * e
        return tuple(carry)

    mean = [None] * R
    rst = [None] * R
    for i in range(R):
        m = _lane_sum(acc[i]) * inv_n
        var = _lane_sum(acc[R + i]) * inv_n - m * m
        mean[i] = m
        rst[i] = _rsqrt(var + EPS)

    @plsc.parallel_loop(0, NV, unroll=4)
    def _norm(j):
        g = gbuf[pl.ds(j * L, L)]
        bb = bbuf[pl.ds(j * L, L)]
        for i in range(R):
            rg = rst[i] * g
            t = bb - mean[i] * rg
            e = ebuf[e0 + i, pl.ds(j * L, L)]
            rows[r0 + i, pl.ds(j * L, L)] = e * rg + t

    return 0


def _body(ids_hbm, wt_hbm, pt_hbm, g_hbm, b_hbm, out_hbm,
          idxb, rows, posb, ebuf, gbuf, bbuf, gsem, ssem):
    wid = lax.axis_index("s") * NC + lax.axis_index("c")
    seq_base = wid * P_PER_W
    inv_n = jnp.float32(1.0 / HIDDEN)

    pltpu.sync_copy(g_hbm, gbuf)
    pltpu.sync_copy(b_hbm, bbuf)

    def pos_base(t):
        return seq_base + (t // BATCH) * K

    def tok_base(t):
        return (t % BATCH) * SEQ + pos_base(t)

    def gather(x):
        buf = x % NBUF
        idx = idxb.at[pl.ds((x % BATCH) * P_PER_W + (x // BATCH) * K, K)]
        pltpu.async_copy(
            wt_hbm.at[idx], rows.at[pl.ds(buf * K, K)], gsem.at[buf])

    for b in range(BATCH):
        pltpu.sync_copy(
            ids_hbm.at[pl.ds(b * SEQ + seq_base, P_PER_W)],
            idxb.at[pl.ds(b * P_PER_W, P_PER_W)])
    pltpu.sync_copy(pt_hbm.at[pl.ds(seq_base, 2 * K)], posb)
    gather(0)
    gather(1)

    def block(t, _):
        buf = t % NBUF

        @pl.when(t < NB - 2)
        def _():
            gather(t + 2)

        pltpu.make_async_copy(
            pt_hbm.at[pl.ds(0, K)], rows.at[pl.ds(buf * K, K)],
            gsem.at[buf]).wait()

        pchunk = (t // BATCH) & 1

        @plsc.parallel_loop(0, K // R)
        def _quads(q):
            _ln_quad(rows, posb, ebuf, gbuf, bbuf,
                     buf * K, pchunk * K, inv_n, q, 0)

        pltpu.async_copy(
            rows.at[pl.ds(buf * K, K)],
            out_hbm.at[pl.ds(tok_base(t), K)], ssem.at[buf])

        @pl.when(t >= 1)
        def _():
            pbuf = (t - 1) % NBUF
            pltpu.make_async_copy(
                out_hbm.at[pl.ds(0, K)], rows.at[pl.ds(pbuf * K, K)],
                ssem.at[pbuf]).wait()

        @pl.when(jnp.logical_and((t + 1) % BATCH == 0, t + 1 < NB))
        def _():
            c1 = (t + 1) // BATCH + 1

            @pl.when(c1 < N_CHUNKS)
            def _():
                pltpu.sync_copy(
                    pt_hbm.at[pl.ds(seq_base + c1 * K, K)],
                    posb.at[pl.ds((c1 & 1) * K, K)])

        return 0

    lax.fori_loop(0, NB, block, 0)
    pltpu.make_async_copy(
        out_hbm.at[pl.ds(0, K)],
        rows.at[pl.ds(((NB - 1) % NBUF) * K, K)],
        ssem.at[(NB - 1) % NBUF]).wait()


@jax.jit
def _run(ids_flat, word_table, pos_table, gamma, beta):
    mesh = plsc.VectorSubcoreMesh(
        core_axis_name="c", subcore_axis_name="s",
        num_cores=NC, num_subcores=NS)
    f = pl.kernel(
        _body,
        out_type=jax.ShapeDtypeStruct((BATCH * SEQ, HIDDEN), jnp.float32),
        mesh=mesh,
        scratch_types=[
            pltpu.VMEM((BATCH * P_PER_W,), jnp.int32),
            pltpu.VMEM((NBUF * K, HIDDEN), jnp.float32),
            pltpu.VMEM((2 * K, HIDDEN), jnp.float32),
            pltpu.VMEM((K, HIDDEN), jnp.float32),
            pltpu.VMEM((HIDDEN,), jnp.float32),
            pltpu.VMEM((HIDDEN,), jnp.float32),
            pltpu.SemaphoreType.DMA((NBUF,)),
            pltpu.SemaphoreType.DMA((NBUF,)),
        ],
    )
    return f(ids_flat, word_table, pos_table, gamma, beta)


def kernel(input_ids, word_table, pos_table, gamma, beta):
    ids_flat = input_ids.reshape(-1).astype(jnp.int32)
    out = _run(ids_flat, word_table, pos_table, gamma, beta)
    return out.reshape(BATCH, SEQ, HIDDEN)


# speedup vs baseline: 1.7959x; 1.7959x over previous
import functools

import jax
import jax.numpy as jnp
from jax import lax
from jax.experimental import pallas as pl
from jax.experimental.pallas import tpu as pltpu
from jax.experimental.pallas import tpu_sc as plsc

VOCAB = 100000
HIDDEN = 1024
MAX_POS = 4096
BATCH = 4
SEQ = 4096
EPS = 1e-12

NC = 2
NS = 16
L = 16
NW = NC * NS
P_PER_W = SEQ // NW
K = 16
N_CHUNKS = P_PER_W // K
NB = N_CHUNKS * BATCH
NV = HIDDEN // L
R = 8
NBUF = 4


def _lane_sum(v):
    idx = lax.iota(jnp.int32, L)
    dnums = lax.GatherDimensionNumbers(
        offset_dims=(), collapsed_slice_dims=(0,), start_index_map=(0,))
    for d in (1, 2, 4, 8):
        perm = lax.gather(
            v, (idx ^ d)[:, None], dnums, slice_sizes=(1,),
            mode=lax.GatherScatterMode.PROMISE_IN_BOUNDS)
        v = v + perm
    return v


def _rsqrt(x):
    yi = lax.bitcast_convert_type(x, jnp.int32)
    yi = 0x5F3759DF - lax.shift_right_logical(yi, 1)
    r = lax.bitcast_convert_type(yi, jnp.float32)
    half_x = 0.5 * x
    for _ in range(3):
        r = r * (1.5 - half_x * r * r)
    return r


def _ln_quad(rows, posb, gbuf, bbuf, base, pbase, inv_n, q, _carry):
    r0 = base + q * R
    p0 = pbase + q * R
    zero = jnp.zeros((L,), jnp.float32)

    @plsc.parallel_loop(0, NV, carry=(zero,) * (2 * R), unroll=4)
    def acc(j, carry):
        carry = list(carry)
        for i in range(R):
            e = rows[r0 + i, pl.ds(j * L, L)] + posb[p0 + i, pl.ds(j * L, L)]
            carry[i] = carry[i] + e
            carry[R + i] = carry[R + i] + e * e
        return tuple(carry)

    mean = [None] * R
    rst = [None] * R
    for i in range(R):
        m = _lane_sum(acc[i]) * inv_n
        var = _lane_sum(acc[R + i]) * inv_n - m * m
        mean[i] = m
        rst[i] = _rsqrt(var + EPS)

    @plsc.parallel_loop(0, NV, unroll=4)
    def _norm(j):
        g = gbuf[pl.ds(j * L, L)]
        bb = bbuf[pl.ds(j * L, L)]
        for i in range(R):
            rg = rst[i] * g
            t = bb - mean[i] * rg
            e = rows[r0 + i, pl.ds(j * L, L)] + posb[p0 + i, pl.ds(j * L, L)]
            rows[r0 + i, pl.ds(j * L, L)] = e * rg + t

    return 0


def _body(ids_hbm, wt_hbm, pt_hbm, g_hbm, b_hbm, out_hbm,
          idxb, rows, posb, gbuf, bbuf, gsem, ssem):
    wid = lax.axis_index("s") * NC + lax.axis_index("c")
    seq_base = wid * P_PER_W
    inv_n = jnp.float32(1.0 / HIDDEN)

    pltpu.sync_copy(g_hbm, gbuf)
    pltpu.sync_copy(b_hbm, bbuf)

    def pos_base(t):
        return seq_base + (t // BATCH) * K

    def tok_base(t):
        return (t % BATCH) * SEQ + pos_base(t)

    def gather(x):
        buf = x % NBUF
        idx = idxb.at[pl.ds((x % BATCH) * P_PER_W + (x // BATCH) * K, K)]
        pltpu.async_copy(
            wt_hbm.at[idx], rows.at[pl.ds(buf * K, K)], gsem.at[buf])

    for b in range(BATCH):
        pltpu.sync_copy(
            ids_hbm.at[pl.ds(b * SEQ + seq_base, P_PER_W)],
            idxb.at[pl.ds(b * P_PER_W, P_PER_W)])
    pltpu.sync_copy(pt_hbm.at[pl.ds(seq_base, 2 * K)], posb)
    gather(0)
    gather(1)

    def block(t, _):
        buf = t % NBUF

        @pl.when(t < NB - 2)
        def _():
            gather(t + 2)

        pltpu.make_async_copy(
            pt_hbm.at[pl.ds(0, K)], rows.at[pl.ds(buf * K, K)],
            gsem.at[buf]).wait()

        pchunk = (t // BATCH) & 1

        @plsc.parallel_loop(0, K // R)
        def _quads(q):
            _ln_quad(rows, posb, gbuf, bbuf, buf * K, pchunk * K, inv_n, q, 0)

        pltpu.async_copy(
            rows.at[pl.ds(buf * K, K)],
            out_hbm.at[pl.ds(tok_base(t), K)], ssem.at[buf])

        @pl.when(t >= 1)
        def _():
            pbuf = (t - 1) % NBUF
            pltpu.make_async_copy(
                out_hbm.at[pl.ds(0, K)], rows.at[pl.ds(pbuf * K, K)],
                ssem.at[pbuf]).wait()

        @pl.when(jnp.logical_and((t + 1) % BATCH == 0, t + 1 < NB))
        def _():
            c1 = (t + 1) // BATCH + 1

            @pl.when(c1 < N_CHUNKS)
            def _():
                pltpu.sync_copy(
                    pt_hbm.at[pl.ds(seq_base + c1 * K, K)],
                    posb.at[pl.ds((c1 & 1) * K, K)])

        return 0

    lax.fori_loop(0, NB, block, 0)
    pltpu.make_async_copy(
        out_hbm.at[pl.ds(0, K)],
        rows.at[pl.ds(((NB - 1) % NBUF) * K, K)],
        ssem.at[(NB - 1) % NBUF]).wait()


@jax.jit
def _run(ids_flat, word_table, pos_table, gamma, beta):
    mesh = plsc.VectorSubcoreMesh(
        core_axis_name="c", subcore_axis_name="s",
        num_cores=NC, num_subcores=NS)
    f = pl.kernel(
        _body,
        out_type=jax.ShapeDtypeStruct((BATCH * SEQ, HIDDEN), jnp.float32),
        mesh=mesh,
        scratch_types=[
            pltpu.VMEM((BATCH * P_PER_W,), jnp.int32),
            pltpu.VMEM((NBUF * K, HIDDEN), jnp.float32),
            pltpu.VMEM((2 * K, HIDDEN), jnp.float32),
            pltpu.VMEM((HIDDEN,), jnp.float32),
            pltpu.VMEM((HIDDEN,), jnp.float32),
            pltpu.SemaphoreType.DMA((NBUF,)),
            pltpu.SemaphoreType.DMA((NBUF,)),
        ],
    )
    return f(ids_flat, word_table, pos_table, gamma, beta)


def kernel(input_ids, word_table, pos_table, gamma, beta):
    ids_flat = input_ids.reshape(-1).astype(jnp.int32)
    out = _run(ids_flat, word_table, pos_table, gamma, beta)
    return out.reshape(BATCH, SEQ, HIDDEN)


# speedup vs baseline: 1.9801x; 1.1026x over previous
import functools

import jax
import jax.numpy as jnp
from jax import lax
from jax.experimental import pallas as pl
from jax.experimental.pallas import tpu as pltpu
from jax.experimental.pallas import tpu_sc as plsc

VOCAB = 100000
HIDDEN = 1024
MAX_POS = 4096
BATCH = 4
SEQ = 4096
EPS = 1e-12

NC = 2
NS = 16
L = 16
NW = NC * NS
P_PER_W = SEQ // NW
K = 16
N_CHUNKS = P_PER_W // K
NB = N_CHUNKS * BATCH
NV = HIDDEN // L
R = 4
NBUF = 4


def _lane_sum(v):
    idx = lax.iota(jnp.int32, L)
    dnums = lax.GatherDimensionNumbers(
        offset_dims=(), collapsed_slice_dims=(0,), start_index_map=(0,))
    for d in (1, 2, 4, 8):
        perm = lax.gather(
            v, (idx ^ d)[:, None], dnums, slice_sizes=(1,),
            mode=lax.GatherScatterMode.PROMISE_IN_BOUNDS)
        v = v + perm
    return v


def _rsqrt(x):
    yi = lax.bitcast_convert_type(x, jnp.int32)
    yi = 0x5F3759DF - lax.shift_right_logical(yi, 1)
    r = lax.bitcast_convert_type(yi, jnp.float32)
    half_x = 0.5 * x
    for _ in range(3):
        r = r * (1.5 - half_x * r * r)
    return r


def _ln_quad(rows, posb, gbuf, bbuf, base, pbase, inv_n, q, _carry):
    r0 = base + q * R
    p0 = pbase + q * R
    zero = jnp.zeros((L,), jnp.float32)

    @plsc.parallel_loop(0, NV, carry=(zero,) * (2 * R), unroll=4)
    def acc(j, carry):
        carry = list(carry)
        for i in range(R):
            e = rows[r0 + i, pl.ds(j * L, L)] + posb[p0 + i, pl.ds(j * L, L)]
            carry[i] = carry[i] + e
            carry[R + i] = carry[R + i] + e * e
        return tuple(carry)

    mean = [None] * R
    rst = [None] * R
    for i in range(R):
        m = _lane_sum(acc[i]) * inv_n
        var = _lane_sum(acc[R + i]) * inv_n - m * m
        mean[i] = m
        rst[i] = _rsqrt(var + EPS)

    @plsc.parallel_loop(0, NV, unroll=4)
    def _norm(j):
        g = gbuf[pl.ds(j * L, L)]
        bb = bbuf[pl.ds(j * L, L)]
        for i in range(R):
            rg = rst[i] * g
            t = bb - mean[i] * rg
            e = rows[r0 + i, pl.ds(j * L, L)] + posb[p0 + i, pl.ds(j * L, L)]
            rows[r0 + i, pl.ds(j * L, L)] = e * rg + t

    return 0


def _body(ids_hbm, wt_hbm, pt_hbm, g_hbm, b_hbm, out_hbm,
          idxb, rows, posb, gbuf, bbuf, gsem, ssem):
    wid = lax.axis_index("s") * NC + lax.axis_index("c")
    seq_base = wid * P_PER_W
    inv_n = jnp.float32(1.0 / HIDDEN)

    pltpu.sync_copy(g_hbm, gbuf)
    pltpu.sync_copy(b_hbm, bbuf)

    def pos_base(t):
        return seq_base + (t // BATCH) * K

    def tok_base(t):
        return (t % BATCH) * SEQ + pos_base(t)

    def gather(x):
        buf = x % NBUF
        idx = idxb.at[pl.ds((x % BATCH) * P_PER_W + (x // BATCH) * K, K)]
        pltpu.async_copy(
            wt_hbm.at[idx], rows.at[pl.ds(buf * K, K)], gsem.at[buf])

    for b in range(BATCH):
        pltpu.sync_copy(
            ids_hbm.at[pl.ds(b * SEQ + seq_base, P_PER_W)],
            idxb.at[pl.ds(b * P_PER_W, P_PER_W)])
    pltpu.sync_copy(pt_hbm.at[pl.ds(seq_base, 2 * K)], posb)
    gather(0)
    gather(1)

    def block(t, _):
        buf = t % NBUF

        @pl.when(t < NB - 2)
        def _():
            gather(t + 2)

        with jax.named_scope("gwait"):
            pltpu.make_async_copy(
                pt_hbm.at[pl.ds(0, K)], rows.at[pl.ds(buf * K, K)],
                gsem.at[buf]).wait()

        pchunk = (t // BATCH) & 1

        with jax.named_scope("ln"):
            @plsc.parallel_loop(0, K // R)
            def _quads(q):
                _ln_quad(rows, posb, gbuf, bbuf,
                         buf * K, pchunk * K, inv_n, q, 0)

        pltpu.async_copy(
            rows.at[pl.ds(buf * K, K)],
            out_hbm.at[pl.ds(tok_base(t), K)], ssem.at[buf])

        with jax.named_scope("swait"):
            @pl.when(t >= 1)
            def _():
                pbuf = (t - 1) % NBUF
                pltpu.make_async_copy(
                    out_hbm.at[pl.ds(0, K)], rows.at[pl.ds(pbuf * K, K)],
                    ssem.at[pbuf]).wait()

        @pl.when(jnp.logical_and((t + 1) % BATCH == 0, t + 1 < NB))
        def _():
            c1 = (t + 1) // BATCH + 1

            @pl.when(c1 < N_CHUNKS)
            def _():
                pltpu.sync_copy(
                    pt_hbm.at[pl.ds(seq_base + c1 * K, K)],
                    posb.at[pl.ds((c1 & 1) * K, K)])

        return 0

    lax.fori_loop(0, NB, block, 0)
    pltpu.make_async_copy(
        out_hbm.at[pl.ds(0, K)],
        rows.at[pl.ds(((NB - 1) % NBUF) * K, K)],
        ssem.at[(NB - 1) % NBUF]).wait()


@jax.jit
def _run(ids_flat, word_table, pos_table, gamma, beta):
    mesh = plsc.VectorSubcoreMesh(
        core_axis_name="c", subcore_axis_name="s",
        num_cores=NC, num_subcores=NS)
    f = pl.kernel(
        _body,
        out_type=jax.ShapeDtypeStruct((BATCH * SEQ, HIDDEN), jnp.float32),
        mesh=mesh,
        scratch_types=[
            pltpu.VMEM((BATCH * P_PER_W,), jnp.int32),
            pltpu.VMEM((NBUF * K, HIDDEN), jnp.float32),
            pltpu.VMEM((2 * K, HIDDEN), jnp.float32),
            pltpu.VMEM((HIDDEN,), jnp.float32),
            pltpu.VMEM((HIDDEN,), jnp.float32),
            pltpu.SemaphoreType.DMA((NBUF,)),
            pltpu.SemaphoreType.DMA((NBUF,)),
        ],
    )
    return f(ids_flat, word_table, pos_table, gamma, beta)


def kernel(input_ids, word_table, pos_table, gamma, beta):
    ids_flat = input_ids.reshape(-1).astype(jnp.int32)
    out = _run(ids_flat, word_table, pos_table, gamma, beta)
    return out.reshape(BATCH, SEQ, HIDDEN)


# speedup vs baseline: 2.1370x; 1.0792x over previous
import functools

import jax
import jax.numpy as jnp
from jax import lax
from jax.experimental import pallas as pl
from jax.experimental.pallas import tpu as pltpu
from jax.experimental.pallas import tpu_sc as plsc

VOCAB = 100000
HIDDEN = 1024
MAX_POS = 4096
BATCH = 4
SEQ = 4096
EPS = 1e-12

NC = 2
NS = 16
L = 16
NW = NC * NS
P_PER_W = SEQ // NW
K = 16
N_CHUNKS = P_PER_W // K
NB = N_CHUNKS * BATCH
NV = HIDDEN // L
R = 4
NBUF = 4


def _lane_sum(v):
    idx = lax.iota(jnp.int32, L)
    dnums = lax.GatherDimensionNumbers(
        offset_dims=(), collapsed_slice_dims=(0,), start_index_map=(0,))
    for d in (1, 2, 4, 8):
        perm = lax.gather(
            v, (idx ^ d)[:, None], dnums, slice_sizes=(1,),
            mode=lax.GatherScatterMode.PROMISE_IN_BOUNDS)
        v = v + perm
    return v


def _rsqrt(x):
    yi = lax.bitcast_convert_type(x, jnp.int32)
    yi = 0x5F3759DF - lax.shift_right_logical(yi, 1)
    r = lax.bitcast_convert_type(yi, jnp.float32)
    half_x = 0.5 * x
    for _ in range(3):
        r = r * (1.5 - half_x * r * r)
    return r


def _ln_quad(rows, posb, gbuf, bbuf, base, pbase, inv_n, q, _carry):
    r0 = base + q * R
    p0 = pbase + q * R
    zero = jnp.zeros((L,), jnp.float32)

    @plsc.parallel_loop(0, NV, carry=(zero,) * (2 * R), unroll=4)
    def acc(j, carry):
        carry = list(carry)
        for i in range(R):
            e = rows[r0 + i, pl.ds(j * L, L)] + posb[p0 + i, pl.ds(j * L, L)]
            carry[i] = carry[i] + e
            carry[R + i] = carry[R + i] + e * e
        return tuple(carry)

    mean = [None] * R
    rst = [None] * R
    for i in range(R):
        m = _lane_sum(acc[i]) * inv_n
        var = _lane_sum(acc[R + i]) * inv_n - m * m
        mean[i] = m
        rst[i] = _rsqrt(var + EPS)

    @plsc.parallel_loop(0, NV, unroll=4)
    def _norm(j):
        g = gbuf[pl.ds(j * L, L)]
        bb = bbuf[pl.ds(j * L, L)]
        for i in range(R):
            rg = rst[i] * g
            t = bb - mean[i] * rg
            e = rows[r0 + i, pl.ds(j * L, L)] + posb[p0 + i, pl.ds(j * L, L)]
            rows[r0 + i, pl.ds(j * L, L)] = e * rg + t

    return 0


def _body(ids_hbm, wt_hbm, pt_hbm, g_hbm, b_hbm, out_hbm,
          idxb, rows, posb, gbuf, bbuf, gsem, ssem, psem):
    wid = lax.axis_index("s") * NC + lax.axis_index("c")
    seq_base = wid * P_PER_W
    inv_n = jnp.float32(1.0 / HIDDEN)

    pltpu.sync_copy(g_hbm, gbuf)
    pltpu.sync_copy(b_hbm, bbuf)

    def pos_base(t):
        return seq_base + (t // BATCH) * K

    def tok_base(t):
        return (t % BATCH) * SEQ + pos_base(t)

    def gather(x):
        buf = x % NBUF
        idx = idxb.at[pl.ds((x % BATCH) * P_PER_W + (x // BATCH) * K, K)]
        pltpu.async_copy(
            wt_hbm.at[idx], rows.at[pl.ds(buf * K, K)], gsem.at[buf])

    for b in range(BATCH):
        pltpu.sync_copy(
            ids_hbm.at[pl.ds(b * SEQ + seq_base, P_PER_W)],
            idxb.at[pl.ds(b * P_PER_W, P_PER_W)])
    pltpu.async_copy(
        pt_hbm.at[pl.ds(seq_base, K)], posb.at[pl.ds(0, K)], psem.at[0])
    pltpu.async_copy(
        pt_hbm.at[pl.ds(seq_base + K, K)], posb.at[pl.ds(K, K)], psem.at[1])
    gather(0)
    gather(1)

    def block(t, _):
        buf = t % NBUF

        @pl.when(t < NB - 2)
        def _():
            gather(t + 2)

        with jax.named_scope("gwait"):
            pltpu.make_async_copy(
                pt_hbm.at[pl.ds(0, K)], rows.at[pl.ds(buf * K, K)],
                gsem.at[buf]).wait()

        pchunk = (t // BATCH) & 1

        with jax.named_scope("pwait"):
            @pl.when(t % BATCH == 0)
            def _():
                pltpu.make_async_copy(
                    pt_hbm.at[pl.ds(0, K)], posb.at[pl.ds(pchunk * K, K)],
                    psem.at[pchunk]).wait()

        with jax.named_scope("ln"):
            @plsc.parallel_loop(0, K // R)
            def _quads(q):
                _ln_quad(rows, posb, gbuf, bbuf,
                         buf * K, pchunk * K, inv_n, q, 0)

        pltpu.async_copy(
            rows.at[pl.ds(buf * K, K)],
            out_hbm.at[pl.ds(tok_base(t), K)], ssem.at[buf])

        with jax.named_scope("swait"):
            @pl.when(t >= 1)
            def _():
                pbuf = (t - 1) % NBUF
                pltpu.make_async_copy(
                    out_hbm.at[pl.ds(0, K)], rows.at[pl.ds(pbuf * K, K)],
                    ssem.at[pbuf]).wait()

        @pl.when(jnp.logical_and((t + 1) % BATCH == 0, t + 1 < NB))
        def _():
            c1 = (t + 1) // BATCH + 1

            @pl.when(c1 < N_CHUNKS)
            def _():
                pltpu.async_copy(
                    pt_hbm.at[pl.ds(seq_base + c1 * K, K)],
                    posb.at[pl.ds((c1 & 1) * K, K)], psem.at[c1 & 1])

        return 0

    lax.fori_loop(0, NB, block, 0)
    pltpu.make_async_copy(
        out_hbm.at[pl.ds(0, K)],
        rows.at[pl.ds(((NB - 1) % NBUF) * K, K)],
        ssem.at[(NB - 1) % NBUF]).wait()


@jax.jit
def _run(ids_flat, word_table, pos_table, gamma, beta):
    mesh = plsc.VectorSubcoreMesh(
        core_axis_name="c", subcore_axis_name="s",
        num_cores=NC, num_subcores=NS)
    f = pl.kernel(
        _body,
        out_type=jax.ShapeDtypeStruct((BATCH * SEQ, HIDDEN), jnp.float32),
        mesh=mesh,
        scratch_types=[
            pltpu.VMEM((BATCH * P_PER_W,), jnp.int32),
            pltpu.VMEM((NBUF * K, HIDDEN), jnp.float32),
            pltpu.VMEM((2 * K, HIDDEN), jnp.float32),
            pltpu.VMEM((HIDDEN,), jnp.float32),
            pltpu.VMEM((HIDDEN,), jnp.float32),
            pltpu.SemaphoreType.DMA((NBUF,)),
            pltpu.SemaphoreType.DMA((NBUF,)),
            pltpu.SemaphoreType.DMA((2,)),
        ],
    )
    return f(ids_flat, word_table, pos_table, gamma, beta)


def kernel(input_ids, word_table, pos_table, gamma, beta):
    ids_flat = input_ids.reshape(-1).astype(jnp.int32)
    out = _run(ids_flat, word_table, pos_table, gamma, beta)
    return out.reshape(BATCH, SEQ, HIDDEN)
